# baseline (speedup 1.0000x reference)
import functools

import jax
import jax.numpy as jnp
from jax import lax
from jax.experimental import pallas as pl
from jax.experimental.pallas import tpu as pltpu
from jax.experimental.pallas import tpu_sc as plsc


@functools.lru_cache(maxsize=None)
def _make_embed(B, D, V):
    info = plsc.get_sparse_core_info()
    NC, NS = info.num_cores, info.num_subcores
    L = info.num_lanes
    NW = NC * NS
    b_per_w = B // NW
    C = 32
    n_chunks = b_per_w // C
    mesh = plsc.VectorSubcoreMesh(core_axis_name="c", subcore_axis_name="s")

    @functools.partial(
        pl.kernel,
        mesh=mesh,
        compiler_params=pltpu.CompilerParams(needs_layout_passes=False),
        out_type=jax.ShapeDtypeStruct((B * D,), jnp.float32),
        scratch_types=[
            pltpu.VMEM((V * D,), jnp.float32),
            pltpu.VMEM((b_per_w,), jnp.int32),
            pltpu.VMEM((2, C * D), jnp.float32),
            pltpu.SemaphoreType.DMA,
            pltpu.SemaphoreType.DMA,
        ],
    )
    def k(table_hbm, idx_hbm, out_hbm, tbl_v, idx_v, rows_v, w0, w1):
        wid = lax.axis_index("s") * NC + lax.axis_index("c")
        base = wid * b_per_w
        wsem = [w0, w1]
        pltpu.sync_copy(table_hbm, tbl_v)
        pltpu.sync_copy(idx_hbm.at[pl.ds(base, b_per_w)], idx_v)
        iota = lax.iota(jnp.int32, L)

        def assemble(i, b):
            def row_body(j, carry):
                jm = j % L
                seg16 = idx_v[pl.ds(i * C + j - jm, L)]
                seg_s = jnp.max(jnp.where(iota == jm, seg16, 0))
                rbase = seg_s * D
                off = j * D
                for g in range(D // L):
                    rows_v[b, pl.ds(off + g * L, L)] = tbl_v[
                        pl.ds(rbase + g * L, L)
                    ]
                return carry

            lax.fori_loop(0, C, row_body, 0)

        def write(i, b):
            return pltpu.async_copy(
                rows_v.at[b],
                out_hbm.at[pl.ds((base + i * C) * D, C * D)],
                wsem[b],
            )

        wh = [None] * n_chunks
        for i in range(n_chunks):
            b = i & 1
            if i >= 2:
                wh[i - 2].wait()
            assemble(i, b)
            wh[i] = write(i, b)
        wh[n_chunks - 1].wait()
        if n_chunks >= 2:
            wh[n_chunks - 2].wait()

    return k


def kernel(segment_ids, table):
    B = segment_ids.shape[0] * segment_ids.shape[1]
    V, D = table.shape
    idx_flat = segment_ids.reshape(B).astype(jnp.int32)
    out = _make_embed(B, D, V)(table.reshape(V * D), idx_flat)
    return out.reshape(segment_ids.shape + (D,))


# speedup vs baseline: 1.7626x; 1.7626x over previous
import functools

import jax
import jax.numpy as jnp
from jax import lax
from jax.experimental import pallas as pl
from jax.experimental.pallas import tpu as pltpu
from jax.experimental.pallas import tpu_sc as plsc


@functools.lru_cache(maxsize=None)
def _make_embed(B, D, V):
    info = plsc.get_sparse_core_info()
    NC, NS = info.num_cores, info.num_subcores
    L = info.num_lanes
    NW = NC * NS
    b_per_w = B // NW
    C = 32
    n_chunks = b_per_w // C
    mesh = plsc.VectorSubcoreMesh(core_axis_name="c", subcore_axis_name="s")

    @functools.partial(
        pl.kernel,
        mesh=mesh,
        compiler_params=pltpu.CompilerParams(needs_layout_passes=False),
        out_type=jax.ShapeDtypeStruct((B * D,), jnp.float32),
        scratch_types=[
            pltpu.VMEM((V * D,), jnp.float32),
            pltpu.VMEM((b_per_w,), jnp.int32),
            pltpu.VMEM((2, C * D), jnp.float32),
            pltpu.SemaphoreType.DMA,
            pltpu.SemaphoreType.DMA,
        ],
    )
    def k(table_hbm, idx_hbm, out_hbm, tbl_v, idx_v, rows_v, w0, w1):
        wid = lax.axis_index("s") * NC + lax.axis_index("c")
        base = wid * b_per_w
        wsem = [w0, w1]
        pltpu.sync_copy(table_hbm, tbl_v)
        pltpu.sync_copy(idx_hbm.at[pl.ds(base, b_per_w)], idx_v)
        iota = lax.iota(jnp.int32, L)

        def assemble(i, b):
            @plsc.parallel_loop(0, C, step=1, unroll=2)
            def row_body(j):
                jm = j % L
                seg16 = idx_v[pl.ds(i * C + j - jm, L)]
                seg_s = jnp.max(jnp.where(iota == jm, seg16, 0))
                rbase = seg_s * D
                off = j * D
                for g in range(D // L):
                    rows_v[b, pl.ds(off + g * L, L)] = tbl_v[
                        pl.ds(rbase + g * L, L)
                    ]

        @pl.loop(0, n_chunks, step=2)
        def chunk_loop(i0):
            for b in range(2):
                i = i0 + b

                @pl.when(i >= 2)
                def _():
                    pltpu.make_async_copy(
                        rows_v.at[b], out_hbm.at[pl.ds(0, C * D)], wsem[b]
                    ).wait()

                assemble(i, b)
                pltpu.async_copy(
                    rows_v.at[b],
                    out_hbm.at[pl.ds((base + i * C) * D, C * D)],
                    wsem[b],
                )

        for b in range(2):
            pltpu.make_async_copy(
                rows_v.at[b], out_hbm.at[pl.ds(0, C * D)], wsem[b]
            ).wait()

    return k


def kernel(segment_ids, table):
    B = segment_ids.shape[0] * segment_ids.shape[1]
    V, D = table.shape
    idx_flat = segment_ids.reshape(B).astype(jnp.int32)
    out = _make_embed(B, D, V)(table.reshape(V * D), idx_flat)
    return out.reshape(segment_ids.shape + (D,))
